# baseline (speedup 1.0000x reference)
import functools

import jax
import jax.numpy as jnp
from jax import lax
from jax.experimental import pallas as pl
from jax.experimental.pallas import tpu as pltpu
from jax.experimental.pallas import tpu_sc as plsc

HID = 768
DLEN = 12
NDIGIT = 13
EPS = 1e-12


_NW = 32
_CHUNK = 64


def _sc_gather(table, idx):
    n = idx.shape[0]
    d = table.shape[1]
    per_w = n // _NW
    nch = per_w // _CHUNK
    mesh = plsc.VectorSubcoreMesh(core_axis_name="c", subcore_axis_name="s")

    @functools.partial(
        pl.kernel,
        mesh=mesh,
        out_type=jax.ShapeDtypeStruct((n, d), jnp.float32),
        scratch_types=[
            pltpu.VMEM((_CHUNK,), jnp.int32),
            pltpu.VMEM((_CHUNK,), jnp.int32),
            pltpu.VMEM((_CHUNK, d), jnp.float32),
            pltpu.VMEM((_CHUNK, d), jnp.float32),
            pltpu.SemaphoreType.DMA,
            pltpu.SemaphoreType.DMA,
        ],
    )
    def gather_kernel(table_hbm, idx_hbm, out_hbm, idx0, idx1, rows0, rows1,
                      sem0, sem1):
        wid = lax.axis_index("s") * 2 + lax.axis_index("c")
        base = wid * per_w
        idx_bufs = (idx0, idx1)
        row_bufs = (rows0, rows1)
        sems = (sem0, sem1)
        pltpu.sync_copy(idx_hbm.at[pl.ds(base, _CHUNK)], idx0)
        copies = [pltpu.async_copy(table_hbm.at[idx0], rows0, sem0)]
        for c in range(nch):
            nxt = c + 1
            if nxt < nch:
                pltpu.sync_copy(
                    idx_hbm.at[pl.ds(base + nxt * _CHUNK, _CHUNK)],
                    idx_bufs[nxt % 2])
                copies.append(
                    pltpu.async_copy(table_hbm.at[idx_bufs[nxt % 2]],
                                     row_bufs[nxt % 2], sems[nxt % 2]))
            copies[c].wait()
            pltpu.sync_copy(row_bufs[c % 2],
                            out_hbm.at[pl.ds(base + c * _CHUNK, _CHUNK)])

    return gather_kernel(table, idx)


_T = 512

_TRANS = (((1,), (1,)), ((), ()))


def _rnn_body(digits_ref, mask_ref, num16_ref, wih_ref, whh_ref, bih_ref,
              bhh_ref, out_ref):
    ctab = (lax.dot_general(num16_ref[...], wih_ref[...], _TRANS,
                            preferred_element_type=jnp.float32)
            + bih_ref[0][None, :] + bhh_ref[0][None, :])

    whh_bf = whh_ref[...].astype(jnp.bfloat16)

    h1tab = jnp.tanh(ctab)
    rec1 = lax.dot_general(h1tab, whh_ref[...], _TRANS,
                           preferred_element_type=jnp.float32)
    h2tab = jnp.tanh(rec1[:, None, :] + ctab[None, :, :]).reshape(256, HID)

    digs = digits_ref[...]
    lanes = lax.broadcasted_iota(jnp.int32, (_T, 16), 1)
    lanes256 = lax.broadcasted_iota(jnp.int32, (_T, 256), 1)

    def ct_for(t):
        oh = (digs[:, t][:, None] == lanes).astype(jnp.float32)
        return jnp.dot(oh, ctab, preferred_element_type=jnp.float32)

    idx2 = digs[:, 0] * 16 + digs[:, 1]
    oh2 = (idx2[:, None] == lanes256).astype(jnp.float32)
    h = jnp.dot(oh2, h2tab, preferred_element_type=jnp.float32)
    for t in range(2, DLEN):
        rec = lax.dot_general(h.astype(jnp.bfloat16), whh_bf, _TRANS,
                              preferred_element_type=jnp.float32)
        h = jnp.tanh(ct_for(t) + rec)

    out_ref[...] = (h * mask_ref[...]).astype(jnp.bfloat16)


def _tc_rnn(digits, mask, num16, w_ih, w_hh, b_ih, b_hh):
    n = digits.shape[0]
    return pl.pallas_call(
        _rnn_body,
        grid=(n // _T,),
        in_specs=[
            pl.BlockSpec((_T, DLEN), lambda i: (i, 0)),
            pl.BlockSpec((_T, 1), lambda i: (i, 0)),
            pl.BlockSpec((16, 32), lambda i: (0, 0)),
            pl.BlockSpec((HID, 32), lambda i: (0, 0)),
            pl.BlockSpec((HID, HID), lambda i: (0, 0)),
            pl.BlockSpec((1, HID), lambda i: (0, 0)),
            pl.BlockSpec((1, HID), lambda i: (0, 0)),
        ],
        out_specs=pl.BlockSpec((_T, HID), lambda i: (i, 0)),
        out_shape=jax.ShapeDtypeStruct((n, HID), jnp.bfloat16),
    )(digits, mask, num16, w_ih, w_hh, b_ih, b_hh)


def _ln_body(wrows_ref, pos_ref, type_ref, lng_ref, lnb_ref, hmask_ref,
             out_ref):
    x = wrows_ref[...] + pos_ref[...] + type_ref[0][None, :]
    mean = jnp.mean(x, axis=-1, keepdims=True)
    cen = x - mean
    var = jnp.mean(cen * cen, axis=-1, keepdims=True)
    ln = cen * lax.rsqrt(var + EPS) * lng_ref[0][None, :] + lnb_ref[0][None, :]
    out_ref[...] = ln + hmask_ref[...].astype(jnp.float32)


def _tc_lnadd(wrows, pos_emb, type_emb, ln_g, ln_b, hmask, nbatch):
    n = wrows.shape[0]
    s = pos_emb.shape[0]
    pos_blocks = s // _T

    def tok_idx(i, j):
        return j * pos_blocks + i

    return pl.pallas_call(
        _ln_body,
        grid=(pos_blocks, nbatch),
        in_specs=[
            pl.BlockSpec((_T, HID), lambda i, j: (tok_idx(i, j), 0)),
            pl.BlockSpec((_T, HID), lambda i, j: (i, 0)),
            pl.BlockSpec((2, HID), lambda i, j: (0, 0)),
            pl.BlockSpec((1, HID), lambda i, j: (0, 0)),
            pl.BlockSpec((1, HID), lambda i, j: (0, 0)),
            pl.BlockSpec((_T, HID), lambda i, j: (tok_idx(i, j), 0)),
        ],
        out_specs=pl.BlockSpec((_T, HID), lambda i, j: (tok_idx(i, j), 0)),
        out_shape=jax.ShapeDtypeStruct((n, HID), jnp.float32),
    )(wrows, pos_emb, type_emb, ln_g, ln_b, hmask)


def kernel(input_ids, digits_ids, number_mask, word_emb, pos_emb, type_emb,
           ln_g, ln_b, num_emb, W_ih, W_hh, b_ih, b_hh):
    bb, ss = input_ids.shape
    n = bb * ss
    digits = digits_ids.reshape(n, DLEN)
    mask = number_mask.reshape(n, 1)
    num16 = jnp.pad(num_emb, ((0, 16 - NDIGIT), (0, 0)))
    hmask = _tc_rnn(digits, mask, num16, W_ih, W_hh,
                    b_ih.reshape(1, HID), b_hh.reshape(1, HID))
    wrows = _sc_gather(word_emb, input_ids.reshape(n))
    out = _tc_lnadd(wrows, pos_emb, type_emb, ln_g.reshape(1, HID),
                    ln_b.reshape(1, HID), hmask, bb)
    return out.reshape(bb, ss, HID)


# speedup vs baseline: 1.0582x; 1.0582x over previous
import functools

import jax
import jax.numpy as jnp
from jax import lax
from jax.experimental import pallas as pl
from jax.experimental.pallas import tpu as pltpu
from jax.experimental.pallas import tpu_sc as plsc

HID = 768
DLEN = 12
NDIGIT = 13
EPS = 1e-12


_NW = 32
_CHUNK = 64


def _sc_gather(table, idx):
    n = idx.shape[0]
    d = table.shape[1]
    per_w = n // _NW
    nch = per_w // _CHUNK
    mesh = plsc.VectorSubcoreMesh(core_axis_name="c", subcore_axis_name="s")

    @functools.partial(
        pl.kernel,
        mesh=mesh,
        out_type=jax.ShapeDtypeStruct((n, d), jnp.float32),
        scratch_types=[
            pltpu.VMEM((_CHUNK,), jnp.int32),
            pltpu.VMEM((_CHUNK,), jnp.int32),
            pltpu.VMEM((_CHUNK, d), jnp.float32),
            pltpu.VMEM((_CHUNK, d), jnp.float32),
            pltpu.SemaphoreType.DMA,
            pltpu.SemaphoreType.DMA,
        ],
    )
    def gather_kernel(table_hbm, idx_hbm, out_hbm, idx0, idx1, rows0, rows1,
                      sem0, sem1):
        wid = lax.axis_index("s") * 2 + lax.axis_index("c")
        base = wid * per_w
        idx_bufs = (idx0, idx1)
        row_bufs = (rows0, rows1)
        sems = (sem0, sem1)
        pltpu.sync_copy(idx_hbm.at[pl.ds(base, _CHUNK)], idx0)
        copies = [pltpu.async_copy(table_hbm.at[idx0], rows0, sem0)]
        for c in range(nch):
            nxt = c + 1
            if nxt < nch:
                pltpu.sync_copy(
                    idx_hbm.at[pl.ds(base + nxt * _CHUNK, _CHUNK)],
                    idx_bufs[nxt % 2])
                copies.append(
                    pltpu.async_copy(table_hbm.at[idx_bufs[nxt % 2]],
                                     row_bufs[nxt % 2], sems[nxt % 2]))
            copies[c].wait()
            pltpu.sync_copy(row_bufs[c % 2],
                            out_hbm.at[pl.ds(base + c * _CHUNK, _CHUNK)])

    return gather_kernel(table, idx)


_T = 512


def _tc_body(wrows_ref, pos_ref, type_ref, lng_ref, lnb_ref, digits_ref,
             mask_ref, num16_ref, wiht_ref, whht_ref, bih_ref, bhh_ref,
             prev_ref, out_ref):
    del prev_ref
    x = wrows_ref[...] + pos_ref[...] + type_ref[0][None, :]
    mean = jnp.mean(x, axis=-1, keepdims=True)
    cen = x - mean
    var = jnp.mean(cen * cen, axis=-1, keepdims=True)
    ln = cen * lax.rsqrt(var + EPS) * lng_ref[0][None, :] + lnb_ref[0][None, :]

    ctab = (jnp.dot(num16_ref[...], wiht_ref[...],
                    preferred_element_type=jnp.float32)
            + bih_ref[0][None, :] + bhh_ref[0][None, :])

    whht_bf = whht_ref[...].astype(jnp.bfloat16)

    h1tab = jnp.tanh(ctab)
    rec1 = jnp.dot(h1tab, whht_ref[...], preferred_element_type=jnp.float32)
    h2tab = jnp.tanh(rec1[:, None, :] + ctab[None, :, :]).reshape(256, HID)

    digs = digits_ref[...]
    lanes = lax.broadcasted_iota(jnp.int32, (_T, 16), 1)
    lanes256 = lax.broadcasted_iota(jnp.int32, (_T, 256), 1)

    def ct_for(t):
        oh = (digs[:, t][:, None] == lanes).astype(jnp.float32)
        return jnp.dot(oh, ctab, preferred_element_type=jnp.float32)

    idx2 = digs[:, 0] * 16 + digs[:, 1]
    oh2 = (idx2[:, None] == lanes256).astype(jnp.float32)
    h = jnp.dot(oh2, h2tab, preferred_element_type=jnp.float32)
    for t in range(2, DLEN):
        rec = jnp.dot(h.astype(jnp.bfloat16), whht_bf,
                      preferred_element_type=jnp.float32)
        h = jnp.tanh(ct_for(t) + rec)

    out_ref[...] = ln + h * mask_ref[...]


def _tc_half(wrows_half, pos_emb, type_emb, ln_g, ln_b, digits, mask, num16,
             w_iht, w_hht, b_ih, b_hh, prev, base):
    n = digits.shape[0]
    s = pos_emb.shape[0]
    nblk = wrows_half.shape[0] // _T
    pos_blocks = s // _T
    return pl.pallas_call(
        _tc_body,
        grid=(nblk,),
        in_specs=[
            pl.BlockSpec((_T, HID), lambda i: (i, 0)),
            pl.BlockSpec((_T, HID), lambda i: ((base + i) % pos_blocks, 0)),
            pl.BlockSpec((2, HID), lambda i: (0, 0)),
            pl.BlockSpec((1, HID), lambda i: (0, 0)),
            pl.BlockSpec((1, HID), lambda i: (0, 0)),
            pl.BlockSpec((_T, DLEN), lambda i: (base + i, 0)),
            pl.BlockSpec((_T, 1), lambda i: (base + i, 0)),
            pl.BlockSpec((16, 32), lambda i: (0, 0)),
            pl.BlockSpec((32, HID), lambda i: (0, 0)),
            pl.BlockSpec((HID, HID), lambda i: (0, 0)),
            pl.BlockSpec((1, HID), lambda i: (0, 0)),
            pl.BlockSpec((1, HID), lambda i: (0, 0)),
            pl.BlockSpec(memory_space=pl.ANY),
        ],
        out_specs=pl.BlockSpec((_T, HID), lambda i: (base + i, 0)),
        out_shape=jax.ShapeDtypeStruct((n, HID), jnp.float32),
        input_output_aliases={12: 0},
    )(wrows_half, pos_emb, type_emb, ln_g, ln_b, digits, mask, num16, w_iht,
      w_hht, b_ih, b_hh, prev)


def kernel(input_ids, digits_ids, number_mask, word_emb, pos_emb, type_emb,
           ln_g, ln_b, num_emb, W_ih, W_hh, b_ih, b_hh):
    bb, ss = input_ids.shape
    n = bb * ss
    half = n // 2
    idx = input_ids.reshape(n)
    w0 = _sc_gather(word_emb, idx[:half])
    w1 = _sc_gather(word_emb, idx[half:])
    digits = digits_ids.reshape(n, DLEN)
    mask = number_mask.reshape(n, 1)
    num16 = jnp.pad(num_emb, ((0, 16 - NDIGIT), (0, 0)))
    lng = ln_g.reshape(1, HID)
    lnb = ln_b.reshape(1, HID)
    bih = b_ih.reshape(1, HID)
    bhh = b_hh.reshape(1, HID)
    w_iht = W_ih.T
    w_hht = W_hh.T
    seed = jnp.zeros((n, HID), dtype=jnp.float32)
    o0 = _tc_half(w0, pos_emb, type_emb, lng, lnb, digits, mask, num16,
                  w_iht, w_hht, bih, bhh, seed, 0)
    out = _tc_half(w1, pos_emb, type_emb, lng, lnb, digits, mask, num16,
                   w_iht, w_hht, bih, bhh, o0, half // _T)
    return out.reshape(bb, ss, HID)


# speedup vs baseline: 1.1652x; 1.1011x over previous
import functools

import jax
import jax.numpy as jnp
from jax import lax
from jax.experimental import pallas as pl
from jax.experimental.pallas import tpu as pltpu
from jax.experimental.pallas import tpu_sc as plsc

HID = 768
DLEN = 12
NDIGIT = 13
EPS = 1e-12


_NW = 32
_CHUNK = 64


def _sc_gather(table, idx):
    n = idx.shape[0]
    d = table.shape[1]
    per_w = n // _NW
    nch = per_w // _CHUNK
    mesh = plsc.VectorSubcoreMesh(core_axis_name="c", subcore_axis_name="s")

    @functools.partial(
        pl.kernel,
        mesh=mesh,
        out_type=jax.ShapeDtypeStruct((n, d), jnp.float32),
        scratch_types=[
            pltpu.VMEM((_CHUNK,), jnp.int32),
            pltpu.VMEM((_CHUNK,), jnp.int32),
            pltpu.VMEM((_CHUNK, d), jnp.float32),
            pltpu.VMEM((_CHUNK, d), jnp.float32),
            pltpu.SemaphoreType.DMA,
            pltpu.SemaphoreType.DMA,
        ],
    )
    def gather_kernel(table_hbm, idx_hbm, out_hbm, idx0, idx1, rows0, rows1,
                      sem0, sem1):
        wid = lax.axis_index("s") * 2 + lax.axis_index("c")
        base = wid * per_w
        idx_bufs = (idx0, idx1)
        row_bufs = (rows0, rows1)
        sems = (sem0, sem1)
        pltpu.sync_copy(idx_hbm.at[pl.ds(base, _CHUNK)], idx0)
        copies = [pltpu.async_copy(table_hbm.at[idx0], rows0, sem0)]
        for c in range(nch):
            nxt = c + 1
            if nxt < nch:
                pltpu.sync_copy(
                    idx_hbm.at[pl.ds(base + nxt * _CHUNK, _CHUNK)],
                    idx_bufs[nxt % 2])
                copies.append(
                    pltpu.async_copy(table_hbm.at[idx_bufs[nxt % 2]],
                                     row_bufs[nxt % 2], sems[nxt % 2]))
            copies[c].wait()
            pltpu.sync_copy(row_bufs[c % 2],
                            out_hbm.at[pl.ds(base + c * _CHUNK, _CHUNK)])

    return gather_kernel(table, idx)


_T = 512


def _tc_body(wrows_ref, pos_ref, type_ref, lng_ref, lnb_ref, digits_ref,
             mask_ref, num16_ref, wiht_ref, whht_ref, bih_ref, bhh_ref,
             prev_ref, out_ref):
    del prev_ref
    x = wrows_ref[...] + pos_ref[...] + type_ref[0][None, :]
    mean = jnp.mean(x, axis=-1, keepdims=True)
    cen = x - mean
    var = jnp.mean(cen * cen, axis=-1, keepdims=True)
    ln = cen * lax.rsqrt(var + EPS) * lng_ref[0][None, :] + lnb_ref[0][None, :]

    ctab = (jnp.dot(num16_ref[...], wiht_ref[...],
                    preferred_element_type=jnp.float32)
            + bih_ref[0][None, :] + bhh_ref[0][None, :])

    whht_bf = whht_ref[...].astype(jnp.bfloat16)

    h1tab = jnp.tanh(ctab)
    rec1 = jnp.dot(h1tab, whht_ref[...], preferred_element_type=jnp.float32)
    h2tab = jnp.tanh(rec1[:, None, :] + ctab[None, :, :]).reshape(256, HID)

    digs = digits_ref[...]
    lanes = lax.broadcasted_iota(jnp.int32, (_T, 16), 1)
    lanes256 = lax.broadcasted_iota(jnp.int32, (_T, 256), 1)

    def ct_for(t):
        oh = (digs[:, t][:, None] == lanes).astype(jnp.float32)
        return jnp.dot(oh, ctab, preferred_element_type=jnp.float32)

    idx2 = digs[:, 0] * 16 + digs[:, 1]
    oh2 = (idx2[:, None] == lanes256).astype(jnp.float32)
    h = jnp.dot(oh2, h2tab, preferred_element_type=jnp.float32)
    for t in range(2, DLEN):
        rec = jnp.dot(h.astype(jnp.bfloat16), whht_bf,
                      preferred_element_type=jnp.float32)
        h = jnp.tanh(ct_for(t) + rec)

    out_ref[...] = ln + h * mask_ref[...]


def _tc_body_first(wrows_ref, pos_ref, type_ref, lng_ref, lnb_ref,
                   digits_ref, mask_ref, num16_ref, wiht_ref, whht_ref,
                   bih_ref, bhh_ref, out_ref):
    _tc_body(wrows_ref, pos_ref, type_ref, lng_ref, lnb_ref, digits_ref,
             mask_ref, num16_ref, wiht_ref, whht_ref, bih_ref, bhh_ref,
             None, out_ref)


def _tc_half(wrows_half, pos_emb, type_emb, ln_g, ln_b, digits, mask, num16,
             w_iht, w_hht, b_ih, b_hh, prev, base):
    n = digits.shape[0]
    s = pos_emb.shape[0]
    nblk = wrows_half.shape[0] // _T
    pos_blocks = s // _T
    in_specs = [
        pl.BlockSpec((_T, HID), lambda i: (i, 0)),
        pl.BlockSpec((_T, HID), lambda i: ((base + i) % pos_blocks, 0)),
        pl.BlockSpec((2, HID), lambda i: (0, 0)),
        pl.BlockSpec((1, HID), lambda i: (0, 0)),
        pl.BlockSpec((1, HID), lambda i: (0, 0)),
        pl.BlockSpec((_T, DLEN), lambda i: (base + i, 0)),
        pl.BlockSpec((_T, 1), lambda i: (base + i, 0)),
        pl.BlockSpec((16, 32), lambda i: (0, 0)),
        pl.BlockSpec((32, HID), lambda i: (0, 0)),
        pl.BlockSpec((HID, HID), lambda i: (0, 0)),
        pl.BlockSpec((1, HID), lambda i: (0, 0)),
        pl.BlockSpec((1, HID), lambda i: (0, 0)),
    ]
    args = [wrows_half, pos_emb, type_emb, ln_g, ln_b, digits, mask, num16,
            w_iht, w_hht, b_ih, b_hh]
    aliases = {}
    body = _tc_body_first
    if prev is not None:
        in_specs.append(pl.BlockSpec(memory_space=pl.ANY))
        args.append(prev)
        aliases = {12: 0}
        body = _tc_body
    return pl.pallas_call(
        body,
        grid=(nblk,),
        in_specs=in_specs,
        out_specs=pl.BlockSpec((_T, HID), lambda i: (base + i, 0)),
        out_shape=jax.ShapeDtypeStruct((n, HID), jnp.float32),
        input_output_aliases=aliases,
    )(*args)


def kernel(input_ids, digits_ids, number_mask, word_emb, pos_emb, type_emb,
           ln_g, ln_b, num_emb, W_ih, W_hh, b_ih, b_hh):
    bb, ss = input_ids.shape
    n = bb * ss
    half = n // 2
    idx = input_ids.reshape(n)
    w0 = _sc_gather(word_emb, idx[:half])
    w1 = _sc_gather(word_emb, idx[half:])
    digits = digits_ids.reshape(n, DLEN)
    mask = number_mask.reshape(n, 1)
    num16 = jnp.pad(num_emb, ((0, 16 - NDIGIT), (0, 0)))
    lng = ln_g.reshape(1, HID)
    lnb = ln_b.reshape(1, HID)
    bih = b_ih.reshape(1, HID)
    bhh = b_hh.reshape(1, HID)
    w_iht = W_ih.T
    w_hht = W_hh.T
    o0 = _tc_half(w0, pos_emb, type_emb, lng, lnb, digits, mask, num16,
                  w_iht, w_hht, bih, bhh, None, 0)
    out = _tc_half(w1, pos_emb, type_emb, lng, lnb, digits, mask, num16,
                   w_iht, w_hht, bih, bhh, o0, half // _T)
    return out.reshape(bb, ss, HID)
